# baseline (speedup 1.0000x reference)
import dataclasses

import jax
import jax.numpy as jnp
from jax.experimental import pallas as pl
from jax.experimental.pallas import tpu as pltpu
from jax.experimental.pallas import tpu_sc as plsc

TOKENS = 256
HIDDEN = 1024
NUM_EXPERTS = 16
FF = 2048
NBUF = 3


def _routing_sc(router_logits):
    mesh = plsc.VectorSubcoreMesh(core_axis_name="c", subcore_axis_name="s")
    cp = pltpu.CompilerParams()
    if "needs_layout_passes" in pltpu.CompilerParams.__dataclass_fields__:
        cp = dataclasses.replace(cp, needs_layout_passes=False)

    @pl.kernel(out_type=jax.ShapeDtypeStruct((TOKENS, NUM_EXPERTS), jnp.float32),
               mesh=mesh, compiler_params=cp,
               scratch_types=[pltpu.VMEM((NUM_EXPERTS,), jnp.float32)])
    def body(rl_hbm, out_hbm, vbuf):
        core = jax.lax.axis_index("c")
        sub = jax.lax.axis_index("s")
        per = TOKENS // 32
        base = (core * 16 + sub) * per

        @pl.loop(0, per)
        def _(t):
            row = base + t
            pltpu.sync_copy(rl_hbm.at[row], vbuf)
            v = vbuf[...]
            m = jnp.max(v)
            p = jnp.exp(v - m)
            idx = jax.lax.broadcasted_iota(jnp.int32, (NUM_EXPERTS,), 0)
            m1 = jnp.max(p)
            i1 = jnp.min(jnp.where(p == m1, idx, NUM_EXPERTS))
            p2 = jnp.where(idx == i1, -1.0, p)
            m2 = jnp.max(p2)
            i2 = jnp.min(jnp.where(p2 == m2, idx, NUM_EXPERTS))
            num = (jnp.where(idx == 2, m1, 0.0) + jnp.where(idx == 3, m2, 0.0))
            den = jnp.where(idx >= 0, m1 + m2, 1.0)
            vbuf[...] = (jnp.where(idx == 0, i1.astype(jnp.float32), 0.0)
                         + jnp.where(idx == 1, i2.astype(jnp.float32), 0.0)
                         + num / den)
            pltpu.sync_copy(vbuf, out_hbm.at[row])

    return body(router_logits)


def _moe_kernel(x_ref, pk_ref, w1_hbm, w3_hbm, w2_hbm, out_ref,
                w1b, w3b, w2b, act_a, act_b, sems,
                i1_ref, i2_ref, g1_ref, g2_ref):
    packed = pk_ref[...]
    i1_ref[...] = packed[:, 0:1].astype(jnp.int32)
    i2_ref[...] = packed[:, 1:2].astype(jnp.int32)
    g1_ref[...] = packed[:, 2:3]
    g2_ref[...] = packed[:, 3:4]

    def c13(e, slot):
        return (
            pltpu.make_async_copy(w1_hbm.at[e], w1b.at[slot], sems.at[slot, 0]),
            pltpu.make_async_copy(w3_hbm.at[e], w3b.at[slot], sems.at[slot, 1]),
        )

    def c2(e, slot):
        return pltpu.make_async_copy(w2_hbm.at[e], w2b.at[slot], sems.at[slot, 2])

    for k in range(NBUF):
        for c in c13(k, k):
            c.start()
        c2(k, k).start()

    xv = x_ref[...]
    out_ref[...] = jnp.zeros((TOKENS, HIDDEN), jnp.float32)

    def swiglu(slot, act_ref):
        h1 = jnp.dot(xv, w1b[slot], preferred_element_type=jnp.float32)
        h3 = jnp.dot(xv, w3b[slot], preferred_element_type=jnp.float32)
        act_ref[...] = ((h1 * jax.lax.logistic(h1)) * h3).astype(jnp.bfloat16)

    def down_acc(e, slot, act_ref):
        y = jnp.dot(act_ref[...], w2b[slot], preferred_element_type=jnp.float32)
        wcol = (jnp.where(i1_ref[...] == e, g1_ref[...], 0.0)
                + jnp.where(i2_ref[...] == e, g2_ref[...], 0.0))
        out_ref[...] += wcol * y

    for c in c13(0, 0):
        c.wait()
    swiglu(0, act_a)
    for c in c13(NBUF, 0):
        c.start()

    def pair(k, _):
        i1_ = 2 * k + 1
        i2_ = 2 * k + 2

        e = i1_ - 1
        slot = jax.lax.rem(i1_, NBUF)
        pslot = jax.lax.rem(e, NBUF)
        for c in c13(i1_, slot):
            c.wait()
        c2(e, pslot).wait()
        swiglu(slot, act_b)
        down_acc(e, pslot, act_a)

        @pl.when(i1_ + NBUF < NUM_EXPERTS)
        def _():
            for c in c13(i1_ + NBUF, slot):
                c.start()

        @pl.when(i1_ + NBUF - 1 < NUM_EXPERTS)
        def _():
            c2(i1_ + NBUF - 1, pslot).start()

        e = i2_ - 1
        slot = jax.lax.rem(i2_, NBUF)
        pslot = jax.lax.rem(e, NBUF)
        for c in c13(i2_, slot):
            c.wait()
        c2(e, pslot).wait()
        swiglu(slot, act_a)
        down_acc(e, pslot, act_b)

        @pl.when(i2_ + NBUF < NUM_EXPERTS)
        def _():
            for c in c13(i2_ + NBUF, slot):
                c.start()

        @pl.when(i2_ + NBUF - 1 < NUM_EXPERTS)
        def _():
            c2(i2_ + NBUF - 1, pslot).start()

        return 0

    jax.lax.fori_loop(0, NUM_EXPERTS // 2 - 1, pair, 0)

    for c in c13(NUM_EXPERTS - 1, (NUM_EXPERTS - 1) % NBUF):
        c.wait()
    c2(NUM_EXPERTS - 2, (NUM_EXPERTS - 2) % NBUF).wait()
    swiglu((NUM_EXPERTS - 1) % NBUF, act_b)
    down_acc(NUM_EXPERTS - 2, (NUM_EXPERTS - 2) % NBUF, act_a)
    c2(NUM_EXPERTS - 1, (NUM_EXPERTS - 1) % NBUF).wait()
    down_acc(NUM_EXPERTS - 1, (NUM_EXPERTS - 1) % NBUF, act_b)


def kernel(x, router_logits, w1, w3, w2):
    packed = _routing_sc(router_logits)
    return pl.pallas_call(
        _moe_kernel,
        in_specs=[
            pl.BlockSpec(memory_space=pltpu.VMEM),
            pl.BlockSpec(memory_space=pltpu.VMEM),
            pl.BlockSpec(memory_space=pltpu.HBM),
            pl.BlockSpec(memory_space=pltpu.HBM),
            pl.BlockSpec(memory_space=pltpu.HBM),
        ],
        out_specs=pl.BlockSpec(memory_space=pltpu.VMEM),
        out_shape=jax.ShapeDtypeStruct((TOKENS, HIDDEN), jnp.float32),
        scratch_shapes=[
            pltpu.VMEM((NBUF, HIDDEN, FF), jnp.bfloat16),
            pltpu.VMEM((NBUF, HIDDEN, FF), jnp.bfloat16),
            pltpu.VMEM((NBUF, FF, HIDDEN), jnp.bfloat16),
            pltpu.VMEM((TOKENS, FF), jnp.bfloat16),
            pltpu.VMEM((TOKENS, FF), jnp.bfloat16),
            pltpu.SemaphoreType.DMA((NBUF, 3)),
            pltpu.VMEM((TOKENS, 1), jnp.int32),
            pltpu.VMEM((TOKENS, 1), jnp.int32),
            pltpu.VMEM((TOKENS, 1), jnp.float32),
            pltpu.VMEM((TOKENS, 1), jnp.float32),
        ],
    )(x, packed, w1, w3, w2)
